# baseline (speedup 1.0000x reference)
import functools

import jax
import jax.numpy as jnp
from jax import lax
from jax.experimental import pallas as pl
from jax.experimental.pallas import tpu as pltpu
from jax.experimental.pallas import tpu_sc as plsc

_B = 4
_S = 8192
_D = 1024
_NW = 32
_S_PER_W = _S // _NW
_CH = 8
_CHW = _CH * _D
_NCHUNK = _S_PER_W // _CH
_NU = _NCHUNK * _B
_L = 16
_UNROLL = 8
_NXB = 8
_NTB = 2
_LOOKAHEAD = 5


def _sc_body(x_hbm, t_hbm, out_hbm, *refs):
    xb = refs[0:_NXB]
    tb = refs[_NXB:_NXB + _NTB]
    xsem = refs[_NXB + _NTB:_NXB + _NTB + _NXB]
    osem = refs[_NXB + _NTB + _NXB:_NXB + _NTB + 2 * _NXB]
    tsem = refs[_NXB + _NTB + 2 * _NXB:]

    wid = lax.axis_index("s") * 2 + lax.axis_index("c")
    word_base = wid * _S_PER_W * _D

    def t_off(ci):
        return word_base + ci * _CHW

    def x_off(u):
        ci, b = divmod(u, _B)
        return b * _S * _D + word_base + ci * _CHW

    def issue_t(ci):
        return pltpu.async_copy(
            t_hbm.at[pl.ds(t_off(ci), _CHW)], tb[ci % _NTB], tsem[ci % _NTB])

    def issue_x(u):
        return pltpu.async_copy(
            x_hbm.at[pl.ds(x_off(u), _CHW)], xb[u % _NXB], xsem[u % _NXB])

    def issue_o(u):
        return pltpu.async_copy(
            xb[u % _NXB], out_hbm.at[pl.ds(x_off(u), _CHW)], osem[u % _NXB])

    pending_t = {0: issue_t(0)}
    pending_x = {u: issue_x(u) for u in range(_LOOKAHEAD)}
    pending_o = {}

    pending_t.pop(0).wait()
    for u in range(_NU):
        pending_x.pop(u).wait()
        pending_o[u] = issue_o(u)
        nxt = u + _LOOKAHEAD
        if nxt < _NU:
            prev = nxt - _NXB
            if prev >= 0:
                pending_o.pop(prev).wait()
            pending_x[nxt] = issue_x(nxt)

    for u in sorted(pending_o):
        pending_o.pop(u).wait()


@functools.partial(jax.jit)
def kernel(x, table):
    xf = x.reshape(_B * _S * _D)
    tf = table.reshape(_S * _D)
    mesh = plsc.VectorSubcoreMesh(core_axis_name="c", subcore_axis_name="s")
    scratch = (
        [pltpu.VMEM((_CHW,), jnp.float32) for _ in range(_NXB)]
        + [pltpu.VMEM((_CHW,), jnp.float32) for _ in range(_NTB)]
        + [pltpu.SemaphoreType.DMA for _ in range(2 * _NXB + _NTB)]
    )
    out = pl.kernel(
        _sc_body,
        mesh=mesh,
        out_type=jax.ShapeDtypeStruct((_B * _S * _D,), jnp.float32),
        scratch_types=scratch,
    )(xf, tf)
    return out.reshape(_B, _S, _D)
